# baseline (speedup 1.0000x reference)
import jax
import jax.numpy as jnp
from jax import lax
from jax.experimental import pallas as pl
from jax.experimental.pallas import tpu as pltpu
from jax.experimental.pallas import tpu_sc as plsc

N = 10000
E = 320000
D = 128

NC = 2
NS = 16
NW = NC * NS

CH = 128
CPW = 80
HALF = CPW // 2
PADDED_E = NW * CPW * CH
EPW = E // NW
PPW = CPW * CH - EPW
XPAD = 256

ROWS_PER_TILE = 624
ZCH = 104
ZREPS = ROWS_PER_TILE // ZCH
ROWS_TAIL = N - NS * ROWS_PER_TILE


def _scale_body(f_ref, n_ref, o_ref):
    o_ref[pl.ds(0, N)] = f_ref[...] * n_ref[...]
    o_ref[pl.ds(N, XPAD)] = jnp.zeros((XPAD, D), jnp.float32)


def _scale(features, norm):
    return pl.pallas_call(
        _scale_body,
        out_shape=jax.ShapeDtypeStruct((N + XPAD, D), jnp.float32),
    )(features, norm)


def _sc_seg_sum_body(x_hbm, src_hbm, dst_hbm, out_hbm,
                     idx_s, idx_d, r0, r1, acc, sg0, sg1, ss0, ss1):
    c = lax.axis_index("c")
    s = lax.axis_index("s")
    w = c * NS + s

    z16 = jnp.zeros((16,), jnp.float32)

    def zrows(i, carry):
        r = i // (D // 16)
        col = (i % (D // 16)) * 16
        r1[r, pl.ds(col, 16)] = z16
        return carry

    lax.fori_loop(0, ZCH * (D // 16), zrows, 0)

    def zacc(k, carry):
        base = s * ROWS_PER_TILE + k * ZCH
        pltpu.sync_copy(r1.at[pl.ds(0, ZCH)], acc.at[pl.ds(base, ZCH)])
        return carry

    lax.fori_loop(0, ZREPS, zacc, 0)

    @pl.when(s == 0)
    def _():
        pltpu.sync_copy(r1.at[pl.ds(0, ROWS_TAIL)],
                        acc.at[pl.ds(NS * ROWS_PER_TILE, ROWS_TAIL)])

    plsc.subcore_barrier()

    def wait_gather(rbuf, sem):
        pltpu.make_async_copy(x_hbm.at[pl.ds(0, CH)], rbuf, sem).wait()

    def body(k, carry):
        j0 = 2 * k
        pltpu.async_copy(x_hbm.at[idx_s.at[j0 + 1]], r1, sg1)
        wait_gather(r0, sg0)
        hs0 = pltpu.async_copy(r0, acc.at[idx_d.at[j0]], ss0, add=True)
        wait_gather(r1, sg1)
        hs0.wait()
        pltpu.async_copy(x_hbm.at[idx_s.at[j0 + 2]], r0, sg0)
        pltpu.async_copy(r1, acc.at[idx_d.at[j0 + 1]], ss1, add=True).wait()
        return carry

    for p in range(CPW // HALF):
        pltpu.sync_copy(src_hbm.at[w, pl.ds(p * HALF, HALF)], idx_s)
        pltpu.sync_copy(dst_hbm.at[w, pl.ds(p * HALF, HALF)], idx_d)
        pltpu.async_copy(x_hbm.at[idx_s.at[0]], r0, sg0)

        lax.fori_loop(0, HALF // 2 - 1, body, 0)

        pltpu.async_copy(x_hbm.at[idx_s.at[HALF - 1]], r1, sg1)
        wait_gather(r0, sg0)
        pltpu.sync_copy(r0, acc.at[idx_d.at[HALF - 2]], add=True)
        wait_gather(r1, sg1)
        pltpu.sync_copy(r1, acc.at[idx_d.at[HALF - 1]], add=True)

    plsc.subcore_barrier()

    def wb(k, carry):
        base = s * ROWS_PER_TILE + k * ZCH
        pltpu.sync_copy(acc.at[pl.ds(base, ZCH)],
                        out_hbm.at[c, pl.ds(base, ZCH)])
        return carry

    lax.fori_loop(0, ZREPS, wb, 0)

    @pl.when(s == 0)
    def _():
        pltpu.sync_copy(acc.at[pl.ds(NS * ROWS_PER_TILE, ROWS_TAIL)],
                        out_hbm.at[c, pl.ds(NS * ROWS_PER_TILE, ROWS_TAIL)])


def _sc_seg_sum(x, src3, dst3):
    mesh = plsc.VectorSubcoreMesh(core_axis_name="c", subcore_axis_name="s")
    return pl.kernel(
        _sc_seg_sum_body,
        mesh=mesh,
        out_type=jax.ShapeDtypeStruct((NC, N, D), jnp.float32),
        scratch_types=[
            pltpu.VMEM((HALF, CH), jnp.int32),
            pltpu.VMEM((HALF, CH), jnp.int32),
            pltpu.VMEM((CH, D), jnp.float32),
            pltpu.VMEM((CH, D), jnp.float32),
            pltpu.VMEM_SHARED((N + XPAD, D), jnp.float32),
            pltpu.SemaphoreType.DMA,
            pltpu.SemaphoreType.DMA,
            pltpu.SemaphoreType.DMA,
            pltpu.SemaphoreType.DMA,
        ],
    )(x, src3, dst3)


def _dense_body(p_ref, f_ref, n_ref, w_ref, b_ref, g_ref, be_ref, o_ref):
    agg = p_ref[0] + p_ref[1]
    h = jnp.dot(agg, w_ref[...], preferred_element_type=jnp.float32)
    h = h + b_ref[...]
    h = h * n_ref[...]
    mean = jnp.mean(h, axis=0, keepdims=True)
    var = jnp.mean((h - mean) ** 2, axis=0, keepdims=True)
    h = (h - mean) / jnp.sqrt(var + 1e-5) * g_ref[...] + be_ref[...]
    h = jnp.maximum(h, 0.0)
    o_ref[...] = f_ref[...] + h


def _dense(partials, features, norm, W, b, gamma, beta):
    return pl.pallas_call(
        _dense_body,
        out_shape=jax.ShapeDtypeStruct((N, D), jnp.float32),
    )(partials, features, norm, W,
      b.reshape(1, D), gamma.reshape(1, D), beta.reshape(1, D))


def kernel(features, edge_index, norm, W, b, gamma, beta):
    ei = edge_index.astype(jnp.int32)
    pad = N + (jnp.arange(NW * PPW, dtype=jnp.int32) % XPAD).reshape(NW, PPW)
    src3 = jnp.concatenate([ei[0].reshape(NW, EPW), pad], axis=1)
    dst3 = jnp.concatenate([ei[1].reshape(NW, EPW), pad], axis=1)
    x = _scale(features, norm)
    partials = _sc_seg_sum(x, src3.reshape(NW, CPW, CH),
                           dst3.reshape(NW, CPW, CH))
    return _dense(partials, features, norm, W, b, gamma, beta)


# speedup vs baseline: 1.0288x; 1.0288x over previous
import jax
import jax.numpy as jnp
from jax import lax
from jax.experimental import pallas as pl
from jax.experimental.pallas import tpu as pltpu
from jax.experimental.pallas import tpu_sc as plsc

N = 10000
E = 320000
D = 128

NC = 2
NS = 16
NW = NC * NS

CH = 128
CPW = 80
HALF = CPW // 2
PADDED_E = NW * CPW * CH
EPW = E // NW
PPW = CPW * CH - EPW
XPAD = 256

ROWS_PER_TILE = 624
ZCH = 104
ZREPS = ROWS_PER_TILE // ZCH
ROWS_TAIL = N - NS * ROWS_PER_TILE


def _scale_body(f_ref, n_ref, o_ref):
    o_ref[pl.ds(0, N)] = f_ref[...] * n_ref[...]
    o_ref[pl.ds(N, XPAD)] = jnp.zeros((XPAD, D), jnp.float32)


def _scale(features, norm):
    return pl.pallas_call(
        _scale_body,
        out_shape=jax.ShapeDtypeStruct((N + XPAD, D), jnp.float32),
    )(features, norm)


def _sc_seg_sum_body(x_hbm, src_hbm, dst_hbm, out_hbm,
                     idx_s, idx_d, r0, r1, acc, sg0, sg1):
    c = lax.axis_index("c")
    s = lax.axis_index("s")
    w = c * NS + s

    z16 = jnp.zeros((16,), jnp.float32)

    def zrows(i, carry):
        r = i // (D // 16)
        col = (i % (D // 16)) * 16
        r1[r, pl.ds(col, 16)] = z16
        return carry

    lax.fori_loop(0, ZCH * (D // 16), zrows, 0)

    def zacc(k, carry):
        base = s * ROWS_PER_TILE + k * ZCH
        pltpu.sync_copy(r1.at[pl.ds(0, ZCH)], acc.at[pl.ds(base, ZCH)])
        return carry

    lax.fori_loop(0, ZREPS, zacc, 0)

    @pl.when(s == 0)
    def _():
        pltpu.sync_copy(r1.at[pl.ds(0, ROWS_TAIL)],
                        acc.at[pl.ds(NS * ROWS_PER_TILE, ROWS_TAIL)])

    plsc.subcore_barrier()

    def wait_gather(rbuf, sem):
        pltpu.make_async_copy(x_hbm.at[pl.ds(0, CH)], rbuf, sem).wait()

    def body(k, carry):
        j0 = 2 * k
        pltpu.async_copy(x_hbm.at[idx_s.at[j0 + 1]], r1, sg1)
        wait_gather(r0, sg0)
        pltpu.sync_copy(r0, acc.at[idx_d.at[j0]], add=True)
        pltpu.async_copy(x_hbm.at[idx_s.at[j0 + 2]], r0, sg0)
        wait_gather(r1, sg1)
        pltpu.sync_copy(r1, acc.at[idx_d.at[j0 + 1]], add=True)
        return carry

    for p in range(CPW // HALF):
        pltpu.sync_copy(src_hbm.at[w, pl.ds(p * HALF, HALF)], idx_s)
        pltpu.sync_copy(dst_hbm.at[w, pl.ds(p * HALF, HALF)], idx_d)
        pltpu.async_copy(x_hbm.at[idx_s.at[0]], r0, sg0)

        lax.fori_loop(0, HALF // 2 - 1, body, 0)

        pltpu.async_copy(x_hbm.at[idx_s.at[HALF - 1]], r1, sg1)
        wait_gather(r0, sg0)
        pltpu.sync_copy(r0, acc.at[idx_d.at[HALF - 2]], add=True)
        wait_gather(r1, sg1)
        pltpu.sync_copy(r1, acc.at[idx_d.at[HALF - 1]], add=True)

    plsc.subcore_barrier()

    def wb(k, carry):
        base = s * ROWS_PER_TILE + k * ZCH
        pltpu.sync_copy(acc.at[pl.ds(base, ZCH)],
                        out_hbm.at[c, pl.ds(base, ZCH)])
        return carry

    lax.fori_loop(0, ZREPS, wb, 0)

    @pl.when(s == 0)
    def _():
        pltpu.sync_copy(acc.at[pl.ds(NS * ROWS_PER_TILE, ROWS_TAIL)],
                        out_hbm.at[c, pl.ds(NS * ROWS_PER_TILE, ROWS_TAIL)])


def _sc_seg_sum(x, src3, dst3):
    mesh = plsc.VectorSubcoreMesh(core_axis_name="c", subcore_axis_name="s")
    return pl.kernel(
        _sc_seg_sum_body,
        mesh=mesh,
        out_type=jax.ShapeDtypeStruct((NC, N, D), jnp.float32),
        scratch_types=[
            pltpu.VMEM((HALF, CH), jnp.int32),
            pltpu.VMEM((HALF, CH), jnp.int32),
            pltpu.VMEM((CH, D), jnp.float32),
            pltpu.VMEM((CH, D), jnp.float32),
            pltpu.VMEM_SHARED((N + XPAD, D), jnp.float32),
            pltpu.SemaphoreType.DMA,
            pltpu.SemaphoreType.DMA,
        ],
    )(x, src3, dst3)


def _dense_body(p_ref, f_ref, n_ref, w_ref, b_ref, g_ref, be_ref, o_ref):
    agg = p_ref[0] + p_ref[1]
    h = jnp.dot(agg, w_ref[...], preferred_element_type=jnp.float32)
    h = h + b_ref[...]
    h = h * n_ref[...]
    mean = jnp.mean(h, axis=0, keepdims=True)
    var = jnp.mean((h - mean) ** 2, axis=0, keepdims=True)
    h = (h - mean) / jnp.sqrt(var + 1e-5) * g_ref[...] + be_ref[...]
    h = jnp.maximum(h, 0.0)
    o_ref[...] = f_ref[...] + h


def _dense(partials, features, norm, W, b, gamma, beta):
    return pl.pallas_call(
        _dense_body,
        out_shape=jax.ShapeDtypeStruct((N, D), jnp.float32),
    )(partials, features, norm, W,
      b.reshape(1, D), gamma.reshape(1, D), beta.reshape(1, D))


def kernel(features, edge_index, norm, W, b, gamma, beta):
    ei = edge_index.astype(jnp.int32)
    pad = N + (jnp.arange(NW * PPW, dtype=jnp.int32) % XPAD).reshape(NW, PPW)
    src3 = jnp.concatenate([ei[0].reshape(NW, EPW), pad], axis=1)
    dst3 = jnp.concatenate([ei[1].reshape(NW, EPW), pad], axis=1)
    x = _scale(features, norm)
    partials = _sc_seg_sum(x, src3.reshape(NW, CPW, CH),
                           dst3.reshape(NW, CPW, CH))
    return _dense(partials, features, norm, W, b, gamma, beta)
